# baseline (speedup 1.0000x reference)
import functools

import jax
import jax.numpy as jnp
from jax import lax
from jax.experimental import pallas as pl
from jax.experimental.pallas import tpu as pltpu
from jax.experimental.pallas import tpu_sc as plsc

_NC = 2
_NS = 16
_NW = _NC * _NS
_CHUNK = 40
_DPAD = 1024


@functools.lru_cache(maxsize=None)
def _make_gather(bf: int, d: int):
    b_per_w = bf // _NW
    n_chunks = b_per_w // _CHUNK
    mesh = plsc.VectorSubcoreMesh(core_axis_name="c", subcore_axis_name="s")

    @functools.partial(
        pl.kernel,
        mesh=mesh,
        out_type=jax.ShapeDtypeStruct((bf, _DPAD), jnp.float32),
        scratch_types=[
            pltpu.VMEM((n_chunks, _CHUNK), jnp.int32),
            pltpu.VMEM((_CHUNK, _DPAD), jnp.float32),
            pltpu.VMEM((_CHUNK, _DPAD), jnp.float32),
            pltpu.SemaphoreType.DMA,
            pltpu.SemaphoreType.DMA,
        ],
    )
    def gather_kernel(table_hbm, idx_hbm, out_hbm, idx_v, rows0, rows1, s0, s1):
        wid = lax.axis_index("s") * _NC + lax.axis_index("c")
        base = wid * b_per_w
        pltpu.sync_copy(idx_hbm.at[wid], idx_v)

        def gather(g, buf, sem):
            pltpu.async_copy(table_hbm.at[idx_v.at[g]], buf, sem)

        def gwait(buf, sem):
            pltpu.make_async_copy(table_hbm.at[idx_v.at[0]], buf, sem).wait()

        def store(g, buf):
            pltpu.sync_copy(buf, out_hbm.at[pl.ds(base + g * _CHUNK, _CHUNK)])

        n_pairs = n_chunks // 2
        gather(0, rows0, s0)

        def body(h, carry):
            g = h * 2
            gather(g + 1, rows1, s1)
            gwait(rows0, s0)
            store(g, rows0)

            @pl.when(h < n_pairs - 1)
            def _():
                gather(g + 2, rows0, s0)

            gwait(rows1, s1)
            store(g + 1, rows1)
            return carry

        lax.fori_loop(0, n_pairs, body, 0)

    return gather_kernel


def kernel(x, prob_table):
    b, l = x.shape
    v, d = prob_table.shape
    bf = b * l
    b_per_w = bf // _NW
    n_chunks = b_per_w // _CHUNK
    table_pad = jnp.pad(prob_table, ((0, 0), (0, _DPAD - d)))
    idx = x.reshape(_NW, n_chunks, _CHUNK).astype(jnp.int32)
    out = _make_gather(bf, d)(table_pad, idx)
    return out[:, :d].reshape(b, l, d)


# speedup vs baseline: 1.0272x; 1.0272x over previous
import functools

import jax
import jax.numpy as jnp
from jax import lax
from jax.experimental import pallas as pl
from jax.experimental.pallas import tpu as pltpu
from jax.experimental.pallas import tpu_sc as plsc

_NC = 2
_NS = 16
_NW = _NC * _NS
_CHUNK = 40


@functools.lru_cache(maxsize=None)
def _make_gather(b: int, l: int, d: int):
    bf = b * l
    b_per_w = bf // _NW
    n_chunks = b_per_w // _CHUNK
    planes_per_chunk = _CHUNK // l
    mesh = plsc.VectorSubcoreMesh(core_axis_name="c", subcore_axis_name="s")

    @functools.partial(
        pl.kernel,
        mesh=mesh,
        out_type=jax.ShapeDtypeStruct((b, l, d), jnp.float32),
        scratch_types=[
            pltpu.VMEM((b_per_w,), jnp.int32),
            pltpu.VMEM((_CHUNK, d), jnp.float32),
            pltpu.VMEM((_CHUNK, d), jnp.float32),
            pltpu.SemaphoreType.DMA,
            pltpu.SemaphoreType.DMA,
        ],
        compiler_params=pltpu.CompilerParams(use_tc_tiling_on_sc=False),
    )
    def gather_kernel(table_hbm, idx_hbm, out_hbm, idx_v, rows0, rows1, s0, s1):
        wid = lax.axis_index("s") * _NC + lax.axis_index("c")
        base_plane = wid * (b_per_w // l)
        pltpu.sync_copy(idx_hbm.at[wid], idx_v)

        def gather(g, buf, sem):
            pltpu.async_copy(
                table_hbm.at[idx_v.at[pl.ds(g * _CHUNK, _CHUNK)]], buf, sem
            )

        def gwait(buf, sem):
            pltpu.make_async_copy(
                table_hbm.at[idx_v.at[pl.ds(0, _CHUNK)]], buf, sem
            ).wait()

        def store(g, buf):
            for p in range(planes_per_chunk):
                pltpu.sync_copy(
                    buf.at[pl.ds(p * l, l)],
                    out_hbm.at[base_plane + g * planes_per_chunk + p],
                )

        n_pairs = n_chunks // 2
        gather(0, rows0, s0)

        def body(h, carry):
            g = h * 2
            gather(g + 1, rows1, s1)
            gwait(rows0, s0)
            store(g, rows0)

            @pl.when(h < n_pairs - 1)
            def _():
                gather(g + 2, rows0, s0)

            gwait(rows1, s1)
            store(g + 1, rows1)
            return carry

        lax.fori_loop(0, n_pairs, body, 0)

    return gather_kernel


def kernel(x, prob_table):
    b, l = x.shape
    v, d = prob_table.shape
    idx = x.reshape(_NW, (b * l) // _NW).astype(jnp.int32)
    return _make_gather(b, l, d)(prob_table, idx)
